# baseline (speedup 1.0000x reference)
import functools
import jax
import jax.numpy as jnp
from jax import lax
from jax.experimental import pallas as pl
from jax.experimental.pallas import tpu as pltpu
from jax.experimental.pallas import tpu_sc as plsc

NC = 2
NS = 16
NW = NC * NS
LANES = 16
CHUNK = 128


def _proj_body(emb_ref, w_ref, rel_ref, t_ref, h_ref):
    r = pl.program_id(1)
    w = w_ref[r]
    p = jnp.dot(emb_ref[...], w, preferred_element_type=jnp.float32)
    t_ref[0] = p
    h_ref[0] = jnp.tanh(p + rel_ref[r])


def _proj_tables(emb_pad, W_r, rel, Npad, D, R, bn):
    nb = Npad // bn
    grid = (nb, R)
    t, h = pl.pallas_call(
        _proj_body,
        grid=grid,
        in_specs=[
            pl.BlockSpec((bn, D), lambda i, r: (i, 0)),
            pl.BlockSpec((R, D, D), lambda i, r: (0, 0, 0)),
            pl.BlockSpec((R, 1, D), lambda i, r: (0, 0, 0)),
        ],
        out_specs=[
            pl.BlockSpec((1, bn, D), lambda i, r: (r, i, 0)),
            pl.BlockSpec((1, bn, D), lambda i, r: (r, i, 0)),
        ],
        out_shape=[
            jax.ShapeDtypeStruct((R, Npad, D), jnp.float32),
            jax.ShapeDtypeStruct((R, Npad, D), jnp.float32),
        ],
    )(emb_pad, W_r, rel.reshape(R, 1, D))
    return t, h


def _attention_kernel(Npad, D, n_chunks):
    mesh = plsc.VectorSubcoreMesh(core_axis_name="c", subcore_axis_name="s")
    rows_per_tile = Npad // NS
    nz = rows_per_tile // CHUNK

    @functools.partial(
        pl.kernel,
        out_type=[
            jax.ShapeDtypeStruct((NW * n_chunks, CHUNK), jnp.float32),
            jax.ShapeDtypeStruct((NC * Npad, LANES), jnp.float32),
        ],
        mesh=mesh,
        compiler_params=pltpu.CompilerParams(needs_layout_passes=False, use_tc_tiling_on_sc=False),
        scratch_types=[
            pltpu.VMEM((n_chunks, CHUNK), jnp.int32),
            pltpu.VMEM((n_chunks, CHUNK), jnp.int32),
            pltpu.VMEM((n_chunks, CHUNK), jnp.float32),
            pltpu.VMEM((CHUNK, 128), jnp.float32),
            pltpu.VMEM((CHUNK, 128), jnp.float32),
            pltpu.VMEM((CHUNK, 128), jnp.float32),
            pltpu.VMEM((CHUNK, 128), jnp.float32),
            pltpu.VMEM((CHUNK, LANES), jnp.float32),
            pltpu.VMEM((CHUNK, LANES), jnp.float32),
            pltpu.VMEM((CHUNK,), jnp.int32),
            pltpu.VMEM((CHUNK,), jnp.int32),
            pltpu.VMEM_SHARED((Npad, LANES), jnp.float32),
            pltpu.SemaphoreType.DMA,
            pltpu.SemaphoreType.DMA,
            pltpu.SemaphoreType.DMA,
            pltpu.SemaphoreType.DMA,
            pltpu.SemaphoreType.DMA,
        ],
    )
    def body(tf, hf, idxt, idxh, dstp, ex_out, den_out,
             iT_all, iH_all, ex_all, trows0, trows1, hrows0, hrows1,
             exrows, zbuf, dstv0, dstv1, den_sh, semt0, semt1, semh0, semh1,
             semsc):
        c = lax.axis_index("c")
        s = lax.axis_index("s")
        wid = s * NC + c
        zero16 = jnp.zeros((LANES,), jnp.float32)
        lanes_iota = lax.iota(jnp.int32, LANES)
        zeros_i = jnp.zeros((LANES,), jnp.int32)

        row0 = wid * n_chunks
        pltpu.sync_copy(idxt.at[pl.ds(row0, n_chunks)], iT_all)
        pltpu.sync_copy(idxh.at[pl.ds(row0, n_chunks)], iH_all)

        def zex(i, _):
            exrows[i, :] = zero16
            zbuf[i, :] = zero16
            return 0
        lax.fori_loop(0, CHUNK, zex, 0)
        for k in range(rows_per_tile // CHUNK):
            pltpu.sync_copy(zbuf, den_sh.at[pl.ds(s * rows_per_tile + k * CHUNK, CHUNK)])
        plsc.subcore_barrier()

        def gather(ch, tbuf, hbuf, dbuf, st, sh):
            pltpu.async_copy(tf.at[iT_all.at[ch]], tbuf, st)
            pltpu.async_copy(dstp.at[row0 + ch], dbuf, st)
            pltpu.async_copy(hf.at[iH_all.at[ch]], hbuf, sh)

        def compute(ch, tbuf, hbuf):
            def group(g, _):
                gbase = g * LANES
                row_idx = lanes_iota + gbase
                av = jnp.zeros((LANES,), jnp.float32)
                for w in range(D):
                    col = jnp.full((LANES,), w, jnp.int32)
                    tcol = plsc.load_gather(tbuf, [row_idx, col])
                    hcol = plsc.load_gather(hbuf, [row_idx, col])
                    av = av + tcol * hcol
                ev = jnp.exp(av)
                ex_all[ch, pl.ds(gbase, LANES)] = ev
                plsc.store_scatter(exrows, [row_idx, zeros_i], ev)
                return 0
            lax.fori_loop(0, CHUNK // LANES, group, 0)

        def scatter(dbuf):
            pltpu.async_copy(exrows, den_sh.at[dbuf], semsc, add=True).wait()

        def wait(ch, tbuf, hbuf, dbuf, st, sh):
            pltpu.make_async_copy(tf.at[iT_all.at[ch]], tbuf, st).wait()
            pltpu.make_async_copy(dstp.at[row0 + ch], dbuf, st).wait()
            pltpu.make_async_copy(hf.at[iH_all.at[ch]], hbuf, sh).wait()

        def chunk_body(ch, _):
            gather(ch, trows0, hrows0, dstv0, semt0, semh0)
            wait(ch, trows0, hrows0, dstv0, semt0, semh0)
            compute(ch, trows0, hrows0)
            scatter(dstv0)
            return 0
        lax.fori_loop(0, n_chunks, chunk_body, 0)

        pltpu.sync_copy(ex_all, ex_out.at[pl.ds(row0, n_chunks)])

        plsc.subcore_barrier()
        r0 = s * rows_per_tile
        pltpu.sync_copy(den_sh.at[pl.ds(r0, rows_per_tile)],
                        den_out.at[pl.ds(c * Npad + r0, rows_per_tile)])

    return body


def _aggregate_kernel(Npad, D, n_chunks):
    mesh = plsc.VectorSubcoreMesh(core_axis_name="c", subcore_axis_name="s")
    nj = D // LANES
    rows_per_tile = Npad // NS
    zrows = 128
    nz = rows_per_tile // zrows

    @functools.partial(
        pl.kernel,
        out_type=jax.ShapeDtypeStruct((NC * Npad, D), jnp.float32),
        mesh=mesh,
        compiler_params=pltpu.CompilerParams(needs_layout_passes=False, use_tc_tiling_on_sc=False),
        scratch_types=[
            pltpu.VMEM((n_chunks, CHUNK), jnp.int32),
            pltpu.VMEM((n_chunks, CHUNK), jnp.float32),
            pltpu.VMEM((CHUNK, D), jnp.float32),
            pltpu.VMEM((CHUNK, D), jnp.float32),
            pltpu.VMEM((zrows, D), jnp.float32),
            pltpu.VMEM((CHUNK,), jnp.int32),
            pltpu.VMEM((CHUNK,), jnp.int32),
            pltpu.VMEM_SHARED((Npad, D), jnp.float32),
            pltpu.SemaphoreType.DMA,
            pltpu.SemaphoreType.DMA,
            pltpu.SemaphoreType.DMA,
        ],
    )
    def body(xt, ex, srcp, dstp, s_out,
             src_all, ex_all, rows0, rows1, zbuf, dstv0, dstv1, s_sh,
             sem0, sem1, semsc):
        c = lax.axis_index("c")
        s = lax.axis_index("s")
        wid = s * NC + c
        zero16 = jnp.zeros((LANES,), jnp.float32)

        row0 = wid * n_chunks
        pltpu.sync_copy(srcp.at[pl.ds(row0, n_chunks)], src_all)
        pltpu.sync_copy(ex.at[pl.ds(row0, n_chunks)], ex_all)

        def zrow(i, _):
            for j in range(nj):
                zbuf[i, pl.ds(j * LANES, LANES)] = zero16
            return 0
        lax.fori_loop(0, zrows, zrow, 0)
        for k in range(nz):
            pltpu.sync_copy(
                zbuf, s_sh.at[pl.ds(s * rows_per_tile + k * zrows, zrows)])
        plsc.subcore_barrier()

        def compute(ch, rbuf):
            def group(g, _):
                gbase = g * LANES
                ev16 = ex_all[ch, pl.ds(gbase, LANES)]
                for k in range(LANES):
                    i = gbase + k
                    e = ev16[k]
                    for j in range(nj):
                        sl = pl.ds(j * LANES, LANES)
                        rbuf[i, sl] = rbuf[i, sl] * e
                return 0
            lax.fori_loop(0, CHUNK // LANES, group, 0)

        def scatter(rbuf, dbuf):
            pltpu.async_copy(rbuf, s_sh.at[dbuf], semsc, add=True).wait()

        def gather(ch, rbuf, dbuf, sem):
            pltpu.async_copy(xt.at[src_all.at[ch]], rbuf, sem)
            pltpu.async_copy(dstp.at[row0 + ch], dbuf, sem)

        def wait(ch, rbuf, dbuf, sem):
            pltpu.make_async_copy(xt.at[src_all.at[ch]], rbuf, sem).wait()
            pltpu.make_async_copy(dstp.at[row0 + ch], dbuf, sem).wait()

        def chunk_body(ch, _):
            gather(ch, rows0, dstv0, sem0)
            wait(ch, rows0, dstv0, sem0)
            compute(ch, rows0)
            scatter(rows0, dstv0)
            return 0
        lax.fori_loop(0, n_chunks, chunk_body, 0)

        plsc.subcore_barrier()
        r0 = s * rows_per_tile
        pltpu.sync_copy(s_sh.at[pl.ds(r0, rows_per_tile)],
                        s_out.at[pl.ds(c * Npad + r0, rows_per_tile)])

    return body


def _layer_body(x_ref, s_ref, den_ref, w1_ref, b1_ref, w2_ref, b2_ref, o_ref):
    den = jnp.sum(den_ref[0] + den_ref[1], axis=-1, keepdims=True)
    ssum = s_ref[0] + s_ref[1]
    h = ssum / (den + 1e-10)
    x = x_ref[...]
    z1 = jnp.dot(x + h, w1_ref[...], preferred_element_type=jnp.float32) + b1_ref[...]
    z2 = jnp.dot(x * h, w2_ref[...], preferred_element_type=jnp.float32) + b2_ref[...]
    o_ref[...] = jnp.where(z1 > 0, z1, 0.01 * z1) + jnp.where(z2 > 0, z2, 0.01 * z2)


def _dense_layer(x, S2, den2, W1, b1, W2, b2, Npad, D, bn):
    nb = Npad // bn
    return pl.pallas_call(
        _layer_body,
        grid=(nb,),
        in_specs=[
            pl.BlockSpec((bn, D), lambda i: (i, 0)),
            pl.BlockSpec((NC, bn, D), lambda i: (0, i, 0)),
            pl.BlockSpec((NC, bn, LANES), lambda i: (0, i, 0)),
            pl.BlockSpec((D, D), lambda i: (0, 0)),
            pl.BlockSpec((1, D), lambda i: (0, 0)),
            pl.BlockSpec((D, D), lambda i: (0, 0)),
            pl.BlockSpec((1, D), lambda i: (0, 0)),
        ],
        out_specs=pl.BlockSpec((bn, D), lambda i: (i, 0)),
        out_shape=jax.ShapeDtypeStruct((Npad, D), jnp.float32),
    )(x, S2, den2, W1, b1.reshape(1, D), W2, b2.reshape(1, D))


def _score_kernel(Npad, D, B):
    mesh = plsc.VectorSubcoreMesh(core_axis_name="c", subcore_axis_name="s")
    nj = D // LANES
    pairs = B // NW

    @functools.partial(
        pl.kernel,
        out_type=jax.ShapeDtypeStruct((B,), jnp.float32),
        mesh=mesh,
        compiler_params=pltpu.CompilerParams(needs_layout_passes=False, use_tc_tiling_on_sc=False),
        scratch_types=[
            pltpu.VMEM((pairs,), jnp.int32),
            pltpu.VMEM((pairs,), jnp.int32),
            pltpu.VMEM((pairs, 128), jnp.float32),
            pltpu.VMEM((pairs, 128), jnp.float32),
            pltpu.VMEM((pairs, 128), jnp.float32),
            pltpu.VMEM((pairs, 128), jnp.float32),
            pltpu.VMEM((pairs, 128), jnp.float32),
            pltpu.VMEM((pairs, 128), jnp.float32),
            pltpu.VMEM((pairs,), jnp.float32),
            pltpu.SemaphoreType.DMA,
            pltpu.SemaphoreType.DMA,
            pltpu.SemaphoreType.DMA,
            pltpu.SemaphoreType.DMA,
            pltpu.SemaphoreType.DMA,
            pltpu.SemaphoreType.DMA,
        ],
    )
    def body(x0, x1, x2, users, items, scores,
             uv, iv, r0u, r0i, r1u, r1i, r2u, r2i, stage,
             s0, s1, s2, s3, s4, s5):
        c = lax.axis_index("c")
        s = lax.axis_index("s")
        wid = s * NC + c
        off = wid * pairs
        pltpu.sync_copy(users.at[pl.ds(off, pairs)], uv)
        pltpu.sync_copy(items.at[pl.ds(off, pairs)], iv)
        cps = [
            pltpu.async_copy(x0.at[uv], r0u, s0),
            pltpu.async_copy(x0.at[iv], r0i, s1),
            pltpu.async_copy(x1.at[uv], r1u, s2),
            pltpu.async_copy(x1.at[iv], r1i, s3),
            pltpu.async_copy(x2.at[uv], r2u, s4),
            pltpu.async_copy(x2.at[iv], r2i, s5),
        ]
        for cp in cps:
            cp.wait()

        lanes_iota = lax.iota(jnp.int32, LANES)

        def group(g, _):
            gbase = g * LANES
            row_idx = lanes_iota + gbase
            av = jnp.zeros((LANES,), jnp.float32)
            for w in range(D):
                col = jnp.full((LANES,), w, jnp.int32)
                av = av + (plsc.load_gather(r0u, [row_idx, col])
                           * plsc.load_gather(r0i, [row_idx, col]))
                av = av + (plsc.load_gather(r1u, [row_idx, col])
                           * plsc.load_gather(r1i, [row_idx, col]))
                av = av + (plsc.load_gather(r2u, [row_idx, col])
                           * plsc.load_gather(r2i, [row_idx, col]))
            stage[pl.ds(gbase, LANES)] = av
            return 0
        lax.fori_loop(0, pairs // LANES, group, 0)
        pltpu.sync_copy(stage, scores.at[pl.ds(off, pairs)])

    return body


def kernel(edge_index, edge_type, users, items, entity_embed, relation_embed,
           W_r, W1_0, b1_0, W2_0, b2_0, W1_1, b1_1, W2_1, b2_1):
    N, D = entity_embed.shape
    R = W_r.shape[0]
    E = edge_type.shape[0]
    B = users.shape[0]

    Npad = ((N + 1023) // 1024) * 1024
    per_tile = ((E + NW * CHUNK - 1) // (NW * CHUNK)) * CHUNK
    Ep = NW * per_tile
    n_chunks = per_tile // CHUNK

    src = edge_index[0].astype(jnp.int32)
    dst = edge_index[1].astype(jnp.int32)
    et = edge_type.astype(jnp.int32)

    pad = Ep - E
    srcp = jnp.concatenate([src, jnp.zeros((pad,), jnp.int32)])
    dstp = jnp.concatenate([dst, jnp.full((pad,), N, jnp.int32)])
    etp = jnp.concatenate([et, jnp.zeros((pad,), jnp.int32)])
    idxT = etp * Npad + srcp
    idxH = etp * Npad + jnp.concatenate([dst, jnp.zeros((pad,), jnp.int32)])

    emb_pad = jnp.pad(entity_embed, ((0, Npad - N), (0, 0)))

    T, H = _proj_tables(emb_pad, W_r, relation_embed, Npad, D, R, bn=1024)
    Tf = T.reshape(R * Npad, D)
    Hf = H.reshape(R * Npad, D)

    nrows = NW * n_chunks
    idxT2 = idxT.reshape(nrows, CHUNK)
    idxH2 = idxH.reshape(nrows, CHUNK)
    srcp2 = srcp.reshape(nrows, CHUNK)
    dstp2 = dstp.reshape(nrows, CHUNK)

    ex2, den = _attention_kernel(Npad, D, n_chunks)(Tf, Hf, idxT2, idxH2, dstp2)
    den2 = den.reshape(NC, Npad, LANES)

    Dh = D // 2
    aggh = _aggregate_kernel(Npad, Dh, n_chunks)

    def aggregate(x):
        slo = aggh(x[:, :Dh], ex2, srcp2, dstp2).reshape(NC, Npad, Dh)
        shi = aggh(x[:, Dh:], ex2, srcp2, dstp2).reshape(NC, Npad, Dh)
        return jnp.concatenate([slo, shi], axis=-1)

    x0 = emb_pad
    S2 = aggregate(x0)
    x1 = _dense_layer(x0, S2, den2, W1_0, b1_0, W2_0, b2_0, Npad, D, bn=1024)
    S2b = aggregate(x1)
    x2 = _dense_layer(x1, S2b, den2, W1_1, b1_1, W2_1, b2_1, Npad, D, bn=1024)

    scores = _score_kernel(Npad, D, B)(
        x0, x1, x2, users.astype(jnp.int32), items.astype(jnp.int32))
    return scores
